# baseline (speedup 1.0000x reference)
import functools

import jax
import jax.numpy as jnp
from jax import lax
from jax.experimental import pallas as pl
from jax.experimental.pallas import tpu as pltpu
from jax.experimental.pallas import tpu_sc as plsc

HIST = 200
D = 64
NIT = 8
DEPTH = 4

_mesh = plsc.VectorSubcoreMesh(
    core_axis_name="c", subcore_axis_name="s", num_cores=2, num_subcores=16
)


def _sc_body(
    uid_hbm, iid_hbm, hist_hbm, user_tt, item_tt,
    w1_hbm, b1_hbm, w2_hbm, b2_hbm, w3_hbm, b3_hbm,
    out_hbm, x1_hbm,
    idx_v, b0, b1x, b2x, b3x, bufU, bufI, parts_v, allp_v, c1p_v,
    w1_v, b1_v, w2_v, b2_v, w3_v, b3f_v,
    h1_v, out_v,
    spart,
    sem_g, sem_u, sem_w1, sem_w2, sem_x,
):
    c = lax.axis_index("c")
    s = lax.axis_index("s")
    wid = s * 2 + c
    active = wid < 25
    finisher = jnp.logical_and(c == 0, s == 0)
    uitile = jnp.logical_and(c == 0, s == 1)
    shipper = jnp.logical_and(c == 1, s == 0)
    iota = lax.iota(jnp.int32, 16)
    bufs = (b0, b1x, b2x, b3x)

    def fetch(table, rid, buf, sem):
        base = pl.multiple_of(rid & -128, 128)
        return pltpu.async_copy(table.at[:, pl.ds(base, 128)], buf, sem)

    w2copies = (
        (w2_hbm, w2_v), (b2_hbm, b2_v), (w3_hbm, w3_v),
    )

    @pl.when(active)
    def _fire_w1():
        pltpu.async_copy(w1_hbm, w1_v, sem_w1)

    @pl.when(finisher)
    def _fire_finish_weights():
        for src, dst in w2copies:
            pltpu.async_copy(src, dst, sem_w2)
        pltpu.async_copy(b3_hbm, b3f_v.at[pl.ds(0, 1)], sem_w2)

    @pl.when(uitile)
    def _fire_ui():
        pltpu.sync_copy(uid_hbm, idx_v.at[pl.ds(16, 1)])
        pltpu.sync_copy(iid_hbm, idx_v.at[pl.ds(24, 1)])
        ivec = idx_v[pl.ds(16, 16)]
        fetch(user_tt, ivec[0], bufU, sem_u)
        fetch(item_tt, ivec[8], bufI, sem_u)
        pltpu.async_copy(b1_hbm, b1_v, sem_u)

    @pl.when(active)
    def _gather_phase():
        pltpu.sync_copy(
            hist_hbm.at[pl.ds(wid * NIT, NIT)], idx_v.at[pl.ds(0, NIT)]
        )
        ivec = idx_v[pl.ds(0, 16)]
        acc = [jnp.zeros((16,), jnp.float32) for _ in range(4)]
        cps = [None] * NIT
        for j in range(DEPTH):
            cps[j] = fetch(item_tt, ivec[j], bufs[j], sem_g)
        for j in range(NIT):
            cps[j].wait()
            lane = jnp.full((16,), ivec[j] & 127, jnp.int32)
            for i in range(4):
                col = plsc.load_gather(bufs[j % DEPTH], [iota + (i * 16), lane])
                acc[i] = acc[i] + col
            if j + DEPTH < NIT:
                cps[j + DEPTH] = fetch(
                    item_tt, ivec[j + DEPTH], bufs[(j + DEPTH) % DEPTH], sem_g
                )
        inv = jnp.float32(1.0 / HIST)
        for i in range(4):
            parts_v[pl.ds(i * 16, 16)] = acc[i] * inv
        pltpu.make_async_copy(w1_hbm, w1_v, sem_w1).wait()

        def mv(tt, h):
            sc = parts_v[pl.ds(tt * 16, 16)]
            for j in range(16):
                sval = sc[j]
                kvec = jnp.full((16,), 128 + tt * 16 + j, jnp.int32)
                h = tuple(
                    h[o] + sval * plsc.load_gather(w1_v, [iota + (o * 16), kvec])
                    for o in range(4)
                )
            return h

        h = lax.fori_loop(
            0, 4, mv, tuple(jnp.zeros((16,), jnp.float32) for _ in range(4))
        )
        for o in range(4):
            parts_v[pl.ds(o * 16, 16)] = h[o]

    @pl.when(jnp.logical_not(active))
    def _zero_phase():
        for o in range(4):
            parts_v[pl.ds(o * 16, 16)] = jnp.zeros((16,), jnp.float32)

    @pl.when(uitile)
    def _ui_phase():
        ivec = idx_v[pl.ds(16, 16)]
        pltpu.make_async_copy(user_tt.at[:, pl.ds(0, 128)], bufU, sem_u).wait()
        pltpu.make_async_copy(item_tt.at[:, pl.ds(0, 128)], bufI, sem_u).wait()
        pltpu.make_async_copy(b1_hbm, b1_v, sem_u).wait()
        ulane = jnp.full((16,), ivec[0] & 127, jnp.int32)
        ilane = jnp.full((16,), ivec[8] & 127, jnp.int32)
        h = tuple(
            parts_v[pl.ds(o * 16, 16)] + b1_v[pl.ds(o * 16, 16)] for o in range(4)
        )
        for base, buf, lanev in ((0, bufU, ulane), (64, bufI, ilane)):
            def mvui(tt, hh, _buf=buf, _lanev=lanev, _base=base):
                ev = plsc.load_gather(_buf, [iota + tt * 16, _lanev])
                for j in range(16):
                    sval = ev[j]
                    kvec = jnp.full((16,), _base + tt * 16 + j, jnp.int32)
                    hh = tuple(
                        hh[o] + sval * plsc.load_gather(w1_v, [iota + (o * 16), kvec])
                        for o in range(4)
                    )
                return hh

            h = lax.fori_loop(0, 4, mvui, h)
        for o in range(4):
            parts_v[pl.ds(o * 16, 16)] = h[o]

    pltpu.sync_copy(parts_v, spart.at[s])
    plsc.subcore_barrier()

    @pl.when(shipper)
    def _core1_reduce():
        pltpu.sync_copy(spart, allp_v)
        acc = [jnp.zeros((16,), jnp.float32) for _ in range(4)]
        for j in range(16):
            for i in range(4):
                acc[i] = acc[i] + allp_v[j, pl.ds(i * 16, 16)]
        for i in range(4):
            parts_v[pl.ds(i * 16, 16)] = acc[i]
        pltpu.sync_copy(parts_v, x1_hbm)
        pltpu.semaphore_signal(sem_x, 1, core_index=0)

    @pl.when(finisher)
    def _finish_phase():
        pltpu.sync_copy(spart, allp_v)
        acc = [jnp.zeros((16,), jnp.float32) for _ in range(4)]
        for j in range(16):
            for i in range(4):
                acc[i] = acc[i] + allp_v[j, pl.ds(i * 16, 16)]
        pl.semaphore_wait(sem_x, 1)
        pltpu.sync_copy(x1_hbm, c1p_v)
        for i in range(4):
            h1_v[pl.ds(i * 16, 16)] = jnp.maximum(
                acc[i] + c1p_v[pl.ds(i * 16, 16)], 0.0
            )
        for src, dst in w2copies:
            pltpu.make_async_copy(src, dst, sem_w2).wait()
        pltpu.make_async_copy(b3_hbm, b3f_v.at[pl.ds(0, 1)], sem_w2).wait()

        def l2(tt, a2):
            hvec = h1_v[pl.ds(tt * 16, 16)]
            for j in range(16):
                sval = hvec[j]
                kvec = jnp.full((16,), tt * 16 + j, jnp.int32)
                a2 = tuple(
                    a2[i] + sval * plsc.load_gather(w2_v, [iota + (i * 16), kvec])
                    for i in range(2)
                )
            return a2

        acc2 = lax.fori_loop(
            0, 4, l2, tuple(b2_v[pl.ds(j * 16, 16)] for j in range(2))
        )
        h2a = jnp.maximum(acc2[0], 0.0)
        h2b = jnp.maximum(acc2[1], 0.0)

        p = h2a * w3_v[0, pl.ds(0, 16)] + h2b * w3_v[0, pl.ds(16, 16)]
        z = b3f_v[...][0]
        for j in range(16):
            z = z + p[j]
        zv = jnp.full((16,), z, jnp.float32)
        out_v[...] = 1.0 / (1.0 + jnp.exp(-zv))
        pltpu.sync_copy(out_v, out_hbm)


_sc_kernel = functools.partial(
    pl.kernel,
    out_type=(
        jax.ShapeDtypeStruct((16,), jnp.float32),
        jax.ShapeDtypeStruct((D,), jnp.float32),
    ),
    mesh=_mesh,
    compiler_params=pltpu.CompilerParams(
        use_tc_tiling_on_sc=True, needs_layout_passes=False
    ),
    scratch_types=[
        pltpu.VMEM((32,), jnp.int32),
        pltpu.VMEM((D, 128), jnp.float32),
        pltpu.VMEM((D, 128), jnp.float32),
        pltpu.VMEM((D, 128), jnp.float32),
        pltpu.VMEM((D, 128), jnp.float32),
        pltpu.VMEM((D, 128), jnp.float32),
        pltpu.VMEM((D, 128), jnp.float32),
        pltpu.VMEM((D,), jnp.float32),
        pltpu.VMEM((16, D), jnp.float32),
        pltpu.VMEM((D,), jnp.float32),
        pltpu.VMEM((64, 192), jnp.float32),
        pltpu.VMEM((64,), jnp.float32),
        pltpu.VMEM((32, 64), jnp.float32),
        pltpu.VMEM((32,), jnp.float32),
        pltpu.VMEM((1, 32), jnp.float32),
        pltpu.VMEM((16,), jnp.float32),
        pltpu.VMEM((64,), jnp.float32),
        pltpu.VMEM((16,), jnp.float32),
        pltpu.VMEM_SHARED((16, D), jnp.float32),
        pltpu.SemaphoreType.DMA,
        pltpu.SemaphoreType.DMA,
        pltpu.SemaphoreType.DMA,
        pltpu.SemaphoreType.DMA,
        pltpu.SemaphoreType.REGULAR,
    ],
)(_sc_body)


def kernel(user_id, item_history, item_id, user_table, item_table, W1, b1, W2, b2, W3, b3):
    out16, _ = _sc_kernel(
        user_id.astype(jnp.int32), item_id.astype(jnp.int32),
        item_history.astype(jnp.int32),
        user_table.T, item_table.T,
        W1, b1, W2, b2, W3, b3,
    )
    return out16[0].reshape(1, 1, 1)


# speedup vs baseline: 1.1248x; 1.1248x over previous
import functools

import jax
import jax.numpy as jnp
from jax import lax
from jax.experimental import pallas as pl
from jax.experimental.pallas import tpu as pltpu
from jax.experimental.pallas import tpu_sc as plsc

HIST = 200
D = 64
NIT = 8
DEPTH = 4

_mesh = plsc.VectorSubcoreMesh(
    core_axis_name="c", subcore_axis_name="s", num_cores=2, num_subcores=16
)


def _sc_body(
    uid_hbm, iid_hbm, hist_hbm, user_tt, item_tt,
    w1_hbm, b1_hbm, w2_hbm, b2_hbm, w3_hbm, b3_hbm,
    out_hbm, x1_hbm,
    idx_v, b0, b1x, b2x, b3x, bufU, bufI, parts_v, allp_v, c1p_v,
    w1_v, b1_v, w2_v, b2_v, w3_v, b3f_v,
    h1_v, out_v,
    spart,
    sem_g, sem_u, sem_w1, sem_w2, sem_x,
):
    c = lax.axis_index("c")
    s = lax.axis_index("s")
    wid = s * 2 + c
    active = jnp.logical_and(wid < 26, wid != 2)
    hidx = wid - (wid > 2).astype(jnp.int32)
    finisher = jnp.logical_and(c == 0, s == 0)
    uitile = jnp.logical_and(c == 0, s == 1)
    shipper = jnp.logical_and(c == 1, s == 0)
    iota = lax.iota(jnp.int32, 16)
    bufs = (b0, b1x, b2x, b3x)

    def fetch(table, rid, buf, sem):
        base = pl.multiple_of(rid & -128, 128)
        return pltpu.async_copy(table.at[:, pl.ds(base, 128)], buf, sem)

    w2copies = (
        (w2_hbm, w2_v), (b2_hbm, b2_v), (w3_hbm, w3_v),
    )

    @pl.when(active)
    def _fire_w1():
        pltpu.async_copy(w1_hbm, w1_v, sem_w1)

    @pl.when(finisher)
    def _fire_finish_weights():
        for src, dst in w2copies:
            pltpu.async_copy(src, dst, sem_w2)
        pltpu.async_copy(b3_hbm, b3f_v.at[pl.ds(0, 1)], sem_w2)

    @pl.when(uitile)
    def _fire_ui():
        pltpu.sync_copy(uid_hbm, idx_v.at[pl.ds(16, 1)])
        pltpu.sync_copy(iid_hbm, idx_v.at[pl.ds(24, 1)])
        ivec = idx_v[pl.ds(16, 16)]
        fetch(user_tt, ivec[0], bufU, sem_u)
        fetch(item_tt, ivec[8], bufI, sem_u)
        pltpu.async_copy(b1_hbm, b1_v, sem_u)

    @pl.when(active)
    def _gather_phase():
        pltpu.sync_copy(
            hist_hbm.at[pl.ds(hidx * NIT, NIT)], idx_v.at[pl.ds(0, NIT)]
        )
        ivec = idx_v[pl.ds(0, 16)]
        acc = [jnp.zeros((16,), jnp.float32) for _ in range(4)]
        cps = [None] * NIT
        for j in range(DEPTH):
            cps[j] = fetch(item_tt, ivec[j], bufs[j], sem_g)
        for j in range(NIT):
            cps[j].wait()
            lane = jnp.full((16,), ivec[j] & 127, jnp.int32)
            for i in range(4):
                col = plsc.load_gather(bufs[j % DEPTH], [iota + (i * 16), lane])
                acc[i] = acc[i] + col
            if j + DEPTH < NIT:
                cps[j + DEPTH] = fetch(
                    item_tt, ivec[j + DEPTH], bufs[(j + DEPTH) % DEPTH], sem_g
                )
        inv = jnp.float32(1.0 / HIST)
        for i in range(4):
            parts_v[pl.ds(i * 16, 16)] = acc[i] * inv
        pltpu.make_async_copy(w1_hbm, w1_v, sem_w1).wait()

        def mv(tt, h):
            sc = parts_v[pl.ds(tt * 16, 16)]
            for j in range(16):
                sval = sc[j]
                kvec = jnp.full((16,), 128 + tt * 16 + j, jnp.int32)
                h = tuple(
                    h[o] + sval * plsc.load_gather(w1_v, [iota + (o * 16), kvec])
                    for o in range(4)
                )
            return h

        h = lax.fori_loop(
            0, 4, mv, tuple(jnp.zeros((16,), jnp.float32) for _ in range(4))
        )
        for o in range(4):
            parts_v[pl.ds(o * 16, 16)] = h[o]

    @pl.when(jnp.logical_and(jnp.logical_not(active), jnp.logical_not(uitile)))
    def _zero_phase():
        for o in range(4):
            parts_v[pl.ds(o * 16, 16)] = jnp.zeros((16,), jnp.float32)

    @pl.when(uitile)
    def _ui_phase():
        ivec = idx_v[pl.ds(16, 16)]
        pltpu.make_async_copy(user_tt.at[:, pl.ds(0, 128)], bufU, sem_u).wait()
        pltpu.make_async_copy(item_tt.at[:, pl.ds(0, 128)], bufI, sem_u).wait()
        pltpu.make_async_copy(b1_hbm, b1_v, sem_u).wait()
        ulane = jnp.full((16,), ivec[0] & 127, jnp.int32)
        ilane = jnp.full((16,), ivec[8] & 127, jnp.int32)
        h = tuple(b1_v[pl.ds(o * 16, 16)] for o in range(4))
        for base, buf, lanev in ((0, bufU, ulane), (64, bufI, ilane)):
            def mvui(tt, hh, _buf=buf, _lanev=lanev, _base=base):
                ev = plsc.load_gather(_buf, [iota + tt * 16, _lanev])
                for j in range(16):
                    sval = ev[j]
                    kvec = jnp.full((16,), _base + tt * 16 + j, jnp.int32)
                    hh = tuple(
                        hh[o] + sval * plsc.load_gather(w1_v, [iota + (o * 16), kvec])
                        for o in range(4)
                    )
                return hh

            h = lax.fori_loop(0, 4, mvui, h)
        for o in range(4):
            parts_v[pl.ds(o * 16, 16)] = h[o]

    pltpu.sync_copy(parts_v, spart.at[s])
    plsc.subcore_barrier()

    @pl.when(shipper)
    def _core1_reduce():
        pltpu.sync_copy(spart, allp_v)
        acc = [jnp.zeros((16,), jnp.float32) for _ in range(4)]
        for j in range(16):
            for i in range(4):
                acc[i] = acc[i] + allp_v[j, pl.ds(i * 16, 16)]
        for i in range(4):
            parts_v[pl.ds(i * 16, 16)] = acc[i]
        pltpu.sync_copy(parts_v, x1_hbm)
        pltpu.semaphore_signal(sem_x, 1, core_index=0)

    @pl.when(finisher)
    def _finish_phase():
        pltpu.sync_copy(spart, allp_v)
        acc = [jnp.zeros((16,), jnp.float32) for _ in range(4)]
        for j in range(16):
            for i in range(4):
                acc[i] = acc[i] + allp_v[j, pl.ds(i * 16, 16)]
        pl.semaphore_wait(sem_x, 1)
        pltpu.sync_copy(x1_hbm, c1p_v)
        for i in range(4):
            h1_v[pl.ds(i * 16, 16)] = jnp.maximum(
                acc[i] + c1p_v[pl.ds(i * 16, 16)], 0.0
            )
        for src, dst in w2copies:
            pltpu.make_async_copy(src, dst, sem_w2).wait()
        pltpu.make_async_copy(b3_hbm, b3f_v.at[pl.ds(0, 1)], sem_w2).wait()

        def l2(tt, a2):
            hvec = h1_v[pl.ds(tt * 16, 16)]
            for j in range(16):
                sval = hvec[j]
                kvec = jnp.full((16,), tt * 16 + j, jnp.int32)
                a2 = tuple(
                    a2[i] + sval * plsc.load_gather(w2_v, [iota + (i * 16), kvec])
                    for i in range(2)
                )
            return a2

        acc2 = lax.fori_loop(
            0, 4, l2, tuple(b2_v[pl.ds(j * 16, 16)] for j in range(2))
        )
        h2a = jnp.maximum(acc2[0], 0.0)
        h2b = jnp.maximum(acc2[1], 0.0)

        p = h2a * w3_v[0, pl.ds(0, 16)] + h2b * w3_v[0, pl.ds(16, 16)]
        z = b3f_v[...][0]
        for j in range(16):
            z = z + p[j]
        zv = jnp.full((16,), z, jnp.float32)
        out_v[...] = 1.0 / (1.0 + jnp.exp(-zv))
        pltpu.sync_copy(out_v, out_hbm)


_sc_kernel = functools.partial(
    pl.kernel,
    out_type=(
        jax.ShapeDtypeStruct((16,), jnp.float32),
        jax.ShapeDtypeStruct((D,), jnp.float32),
    ),
    mesh=_mesh,
    compiler_params=pltpu.CompilerParams(
        use_tc_tiling_on_sc=True, needs_layout_passes=False
    ),
    scratch_types=[
        pltpu.VMEM((32,), jnp.int32),
        pltpu.VMEM((D, 128), jnp.float32),
        pltpu.VMEM((D, 128), jnp.float32),
        pltpu.VMEM((D, 128), jnp.float32),
        pltpu.VMEM((D, 128), jnp.float32),
        pltpu.VMEM((D, 128), jnp.float32),
        pltpu.VMEM((D, 128), jnp.float32),
        pltpu.VMEM((D,), jnp.float32),
        pltpu.VMEM((16, D), jnp.float32),
        pltpu.VMEM((D,), jnp.float32),
        pltpu.VMEM((64, 192), jnp.float32),
        pltpu.VMEM((64,), jnp.float32),
        pltpu.VMEM((32, 64), jnp.float32),
        pltpu.VMEM((32,), jnp.float32),
        pltpu.VMEM((1, 32), jnp.float32),
        pltpu.VMEM((16,), jnp.float32),
        pltpu.VMEM((64,), jnp.float32),
        pltpu.VMEM((16,), jnp.float32),
        pltpu.VMEM_SHARED((16, D), jnp.float32),
        pltpu.SemaphoreType.DMA,
        pltpu.SemaphoreType.DMA,
        pltpu.SemaphoreType.DMA,
        pltpu.SemaphoreType.DMA,
        pltpu.SemaphoreType.REGULAR,
    ],
)(_sc_body)


def kernel(user_id, item_history, item_id, user_table, item_table, W1, b1, W2, b2, W3, b3):
    out16, _ = _sc_kernel(
        user_id.astype(jnp.int32), item_id.astype(jnp.int32),
        item_history.astype(jnp.int32),
        user_table.T, item_table.T,
        W1, b1, W2, b2, W3, b3,
    )
    return out16[0].reshape(1, 1, 1)


# speedup vs baseline: 1.1278x; 1.0027x over previous
import functools

import jax
import jax.numpy as jnp
from jax import lax
from jax.experimental import pallas as pl
from jax.experimental.pallas import tpu as pltpu
from jax.experimental.pallas import tpu_sc as plsc

HIST = 200
D = 64
NIT = 8
DEPTH = 4

_mesh = plsc.VectorSubcoreMesh(
    core_axis_name="c", subcore_axis_name="s", num_cores=2, num_subcores=16
)


def _sc_body(
    uid_hbm, iid_hbm, hist_hbm, user_tt, item_tt,
    w1_hbm, b1_hbm, w2_hbm, b2_hbm, w3_hbm, b3_hbm,
    out_hbm, x1_hbm,
    idx_v, b0, b1x, b2x, b3x, bufU, bufI, parts_v, allp_v, c1p_v,
    w1_v, b1_v, w2_v, b2_v, w3_v, b3f_v,
    h1_v, out_v,
    spart,
    sem_g, sem_u, sem_w1, sem_w2, sem_x,
):
    c = lax.axis_index("c")
    s = lax.axis_index("s")
    wid = s * 2 + c
    active = wid < 25
    finisher = jnp.logical_and(c == 0, s == 0)
    uitile = jnp.logical_and(c == 1, s == 12)
    shipper = jnp.logical_and(c == 1, s == 0)
    iota = lax.iota(jnp.int32, 16)
    bufs = (b0, b1x, b2x, b3x)

    def fetch(table, rid, buf, sem):
        base = pl.multiple_of(rid & -128, 128)
        return pltpu.async_copy(table.at[:, pl.ds(base, 128)], buf, sem)

    w2copies = (
        (w2_hbm, w2_v), (b2_hbm, b2_v), (w3_hbm, w3_v),
    )

    @pl.when(active)
    def _fire_w1():
        pltpu.async_copy(w1_hbm, w1_v, sem_w1)

    @pl.when(finisher)
    def _fire_finish_weights():
        for src, dst in w2copies:
            pltpu.async_copy(src, dst, sem_w2)
        pltpu.async_copy(b3_hbm, b3f_v.at[pl.ds(0, 1)], sem_w2)

    @pl.when(uitile)
    def _fire_ui():
        pltpu.sync_copy(uid_hbm, idx_v.at[pl.ds(16, 1)])
        pltpu.sync_copy(iid_hbm, idx_v.at[pl.ds(24, 1)])
        ivec = idx_v[pl.ds(16, 16)]
        fetch(user_tt, ivec[0], bufU, sem_u)
        fetch(item_tt, ivec[8], bufI, sem_u)
        pltpu.async_copy(b1_hbm, b1_v, sem_u)

    @pl.when(active)
    def _gather_phase():
        pltpu.sync_copy(
            hist_hbm.at[pl.ds(wid * NIT, NIT)], idx_v.at[pl.ds(0, NIT)]
        )
        ivec = idx_v[pl.ds(0, 16)]
        acc = [jnp.zeros((16,), jnp.float32) for _ in range(4)]
        cps = [None] * NIT
        for j in range(DEPTH):
            cps[j] = fetch(item_tt, ivec[j], bufs[j], sem_g)
        for j in range(NIT):
            cps[j].wait()
            lane = jnp.full((16,), ivec[j] & 127, jnp.int32)
            for i in range(4):
                col = plsc.load_gather(bufs[j % DEPTH], [iota + (i * 16), lane])
                acc[i] = acc[i] + col
            if j + DEPTH < NIT:
                cps[j + DEPTH] = fetch(
                    item_tt, ivec[j + DEPTH], bufs[(j + DEPTH) % DEPTH], sem_g
                )
        inv = jnp.float32(1.0 / HIST)
        for i in range(4):
            parts_v[pl.ds(i * 16, 16)] = acc[i] * inv
        pltpu.make_async_copy(w1_hbm, w1_v, sem_w1).wait()

        def mv(tt, h):
            sc = parts_v[pl.ds(tt * 16, 16)]
            for j in range(16):
                sval = sc[j]
                kvec = jnp.full((16,), 128 + tt * 16 + j, jnp.int32)
                h = tuple(
                    h[o] + sval * plsc.load_gather(w1_v, [iota + (o * 16), kvec])
                    for o in range(4)
                )
            return h

        h = lax.fori_loop(
            0, 4, mv, tuple(jnp.zeros((16,), jnp.float32) for _ in range(4))
        )
        for o in range(4):
            parts_v[pl.ds(o * 16, 16)] = h[o]

    @pl.when(jnp.logical_and(jnp.logical_not(active), jnp.logical_not(uitile)))
    def _zero_phase():
        for o in range(4):
            parts_v[pl.ds(o * 16, 16)] = jnp.zeros((16,), jnp.float32)

    @pl.when(uitile)
    def _ui_phase():
        ivec = idx_v[pl.ds(16, 16)]
        pltpu.make_async_copy(user_tt.at[:, pl.ds(0, 128)], bufU, sem_u).wait()
        pltpu.make_async_copy(item_tt.at[:, pl.ds(0, 128)], bufI, sem_u).wait()
        pltpu.make_async_copy(b1_hbm, b1_v, sem_u).wait()
        ulane = jnp.full((16,), ivec[0] & 127, jnp.int32)
        ilane = jnp.full((16,), ivec[8] & 127, jnp.int32)
        h = tuple(b1_v[pl.ds(o * 16, 16)] for o in range(4))
        for base, buf, lanev in ((0, bufU, ulane), (64, bufI, ilane)):
            def mvui(tt, hh, _buf=buf, _lanev=lanev, _base=base):
                ev = plsc.load_gather(_buf, [iota + tt * 16, _lanev])
                for j in range(16):
                    sval = ev[j]
                    kvec = jnp.full((16,), _base + tt * 16 + j, jnp.int32)
                    hh = tuple(
                        hh[o] + sval * plsc.load_gather(w1_v, [iota + (o * 16), kvec])
                        for o in range(4)
                    )
                return hh

            h = lax.fori_loop(0, 4, mvui, h)
        for o in range(4):
            parts_v[pl.ds(o * 16, 16)] = h[o]

    pltpu.sync_copy(parts_v, spart.at[s])
    plsc.subcore_barrier()

    @pl.when(shipper)
    def _core1_reduce():
        pltpu.sync_copy(spart, allp_v)
        acc = [jnp.zeros((16,), jnp.float32) for _ in range(4)]
        for j in range(16):
            for i in range(4):
                acc[i] = acc[i] + allp_v[j, pl.ds(i * 16, 16)]
        for i in range(4):
            parts_v[pl.ds(i * 16, 16)] = acc[i]
        pltpu.sync_copy(parts_v, x1_hbm)
        pltpu.semaphore_signal(sem_x, 1, core_index=0)

    @pl.when(finisher)
    def _finish_phase():
        pltpu.sync_copy(spart, allp_v)
        acc = [jnp.zeros((16,), jnp.float32) for _ in range(4)]
        for j in range(16):
            for i in range(4):
                acc[i] = acc[i] + allp_v[j, pl.ds(i * 16, 16)]
        pl.semaphore_wait(sem_x, 1)
        pltpu.sync_copy(x1_hbm, c1p_v)
        for i in range(4):
            h1_v[pl.ds(i * 16, 16)] = jnp.maximum(
                acc[i] + c1p_v[pl.ds(i * 16, 16)], 0.0
            )
        for src, dst in w2copies:
            pltpu.make_async_copy(src, dst, sem_w2).wait()
        pltpu.make_async_copy(b3_hbm, b3f_v.at[pl.ds(0, 1)], sem_w2).wait()

        def l2(tt, a2):
            hvec = h1_v[pl.ds(tt * 16, 16)]
            for j in range(16):
                sval = hvec[j]
                kvec = jnp.full((16,), tt * 16 + j, jnp.int32)
                a2 = tuple(
                    a2[i] + sval * plsc.load_gather(w2_v, [iota + (i * 16), kvec])
                    for i in range(2)
                )
            return a2

        acc2 = lax.fori_loop(
            0, 4, l2, tuple(b2_v[pl.ds(j * 16, 16)] for j in range(2))
        )
        h2a = jnp.maximum(acc2[0], 0.0)
        h2b = jnp.maximum(acc2[1], 0.0)

        p = h2a * w3_v[0, pl.ds(0, 16)] + h2b * w3_v[0, pl.ds(16, 16)]
        z = b3f_v[...][0]
        for j in range(16):
            z = z + p[j]
        zv = jnp.full((16,), z, jnp.float32)
        out_v[...] = 1.0 / (1.0 + jnp.exp(-zv))
        pltpu.sync_copy(out_v, out_hbm)


_sc_kernel = functools.partial(
    pl.kernel,
    out_type=(
        jax.ShapeDtypeStruct((16,), jnp.float32),
        jax.ShapeDtypeStruct((D,), jnp.float32),
    ),
    mesh=_mesh,
    compiler_params=pltpu.CompilerParams(
        use_tc_tiling_on_sc=True, needs_layout_passes=False
    ),
    scratch_types=[
        pltpu.VMEM((32,), jnp.int32),
        pltpu.VMEM((D, 128), jnp.float32),
        pltpu.VMEM((D, 128), jnp.float32),
        pltpu.VMEM((D, 128), jnp.float32),
        pltpu.VMEM((D, 128), jnp.float32),
        pltpu.VMEM((D, 128), jnp.float32),
        pltpu.VMEM((D, 128), jnp.float32),
        pltpu.VMEM((D,), jnp.float32),
        pltpu.VMEM((16, D), jnp.float32),
        pltpu.VMEM((D,), jnp.float32),
        pltpu.VMEM((64, 192), jnp.float32),
        pltpu.VMEM((64,), jnp.float32),
        pltpu.VMEM((32, 64), jnp.float32),
        pltpu.VMEM((32,), jnp.float32),
        pltpu.VMEM((1, 32), jnp.float32),
        pltpu.VMEM((16,), jnp.float32),
        pltpu.VMEM((64,), jnp.float32),
        pltpu.VMEM((16,), jnp.float32),
        pltpu.VMEM_SHARED((16, D), jnp.float32),
        pltpu.SemaphoreType.DMA,
        pltpu.SemaphoreType.DMA,
        pltpu.SemaphoreType.DMA,
        pltpu.SemaphoreType.DMA,
        pltpu.SemaphoreType.REGULAR,
    ],
)(_sc_body)


def kernel(user_id, item_history, item_id, user_table, item_table, W1, b1, W2, b2, W3, b3):
    out16, _ = _sc_kernel(
        user_id.astype(jnp.int32), item_id.astype(jnp.int32),
        item_history.astype(jnp.int32),
        user_table.T, item_table.T,
        W1, b1, W2, b2, W3, b3,
    )
    return out16[0].reshape(1, 1, 1)


# speedup vs baseline: 1.1284x; 1.0006x over previous
import functools

import jax
import jax.numpy as jnp
from jax import lax
from jax.experimental import pallas as pl
from jax.experimental.pallas import tpu as pltpu
from jax.experimental.pallas import tpu_sc as plsc

HIST = 200
D = 64
NIT = 8
DEPTH = 4

_mesh = plsc.VectorSubcoreMesh(
    core_axis_name="c", subcore_axis_name="s", num_cores=2, num_subcores=16
)


def _sc_body(
    uid_hbm, iid_hbm, hist_hbm, user_tt, item_tt,
    w1_hbm, b1_hbm, w2_hbm, b2_hbm, w3_hbm, b3_hbm,
    out_hbm, x1_hbm,
    idx_v, b0, b1x, b2x, b3x, b4, b5, b6, b7, bufU, bufI, parts_v, allp_v, c1p_v,
    w1_v, b1_v, w2_v, b2_v, w3_v, b3f_v,
    h1_v, out_v,
    spart,
    sem_g, sem_u, sem_w1, sem_w2, sem_x,
):
    c = lax.axis_index("c")
    s = lax.axis_index("s")
    wid = s * 2 + c
    active = wid < 25
    finisher = jnp.logical_and(c == 0, s == 0)
    uitile = jnp.logical_and(c == 1, s == 12)
    shipper = jnp.logical_and(c == 1, s == 0)
    iota = lax.iota(jnp.int32, 16)
    bufs = (b0, b1x, b2x, b3x, b4, b5, b6, b7)

    def fetch(table, rid, buf, sem):
        base = pl.multiple_of(rid & -128, 128)
        return pltpu.async_copy(table.at[:, pl.ds(base, 128)], buf, sem)

    w2copies = (
        (w2_hbm, w2_v), (b2_hbm, b2_v), (w3_hbm, w3_v),
    )

    @pl.when(jnp.logical_or(active, uitile))
    def _fire_w1():
        pltpu.async_copy(w1_hbm, w1_v, sem_w1)

    @pl.when(finisher)
    def _fire_finish_weights():
        for src, dst in w2copies:
            pltpu.async_copy(src, dst, sem_w2)
        pltpu.async_copy(b3_hbm, b3f_v.at[pl.ds(0, 1)], sem_w2)

    @pl.when(uitile)
    def _fire_ui():
        pltpu.sync_copy(uid_hbm, idx_v.at[pl.ds(16, 1)])
        pltpu.sync_copy(iid_hbm, idx_v.at[pl.ds(32, 1)])
        ivec = idx_v[pl.ds(16, 16)]
        ivec2 = idx_v[pl.ds(32, 16)]
        fetch(user_tt, ivec[0], bufU, sem_u)
        fetch(item_tt, ivec2[0], bufI, sem_u)
        pltpu.async_copy(b1_hbm, b1_v, sem_u)

    @pl.when(active)
    def _gather_phase():
        pltpu.sync_copy(
            hist_hbm.at[pl.ds(wid * NIT, NIT)], idx_v.at[pl.ds(0, NIT)]
        )
        ivec = idx_v[pl.ds(0, 16)]
        rids = [ivec[j] for j in range(NIT)]
        acc = [jnp.zeros((16,), jnp.float32) for _ in range(4)]
        cps = [fetch(item_tt, rids[j], bufs[j], sem_g) for j in range(NIT)]
        for cp in cps:
            cp.wait()
        for j in range(NIT):
            lane = jnp.full((16,), rids[j] & 127, jnp.int32)
            for i in range(4):
                col = plsc.load_gather(bufs[j], [iota + (i * 16), lane])
                acc[i] = acc[i] + col
        inv = jnp.float32(1.0 / HIST)
        for i in range(4):
            parts_v[pl.ds(i * 16, 16)] = acc[i] * inv
        pltpu.make_async_copy(w1_hbm, w1_v, sem_w1).wait()

        def mv(tt, h):
            sc = parts_v[pl.ds(tt * 16, 16)]
            for j in range(16):
                sval = sc[j]
                kvec = jnp.full((16,), 128 + tt * 16 + j, jnp.int32)
                h = tuple(
                    h[o] + sval * plsc.load_gather(w1_v, [iota + (o * 16), kvec])
                    for o in range(4)
                )
            return h

        h = lax.fori_loop(
            0, 4, mv, tuple(jnp.zeros((16,), jnp.float32) for _ in range(4))
        )
        for o in range(4):
            parts_v[pl.ds(o * 16, 16)] = h[o]

    @pl.when(jnp.logical_and(jnp.logical_not(active), jnp.logical_not(uitile)))
    def _zero_phase():
        for o in range(4):
            parts_v[pl.ds(o * 16, 16)] = jnp.zeros((16,), jnp.float32)

    @pl.when(uitile)
    def _ui_phase():
        ivec = idx_v[pl.ds(16, 16)]
        ivec2 = idx_v[pl.ds(32, 16)]
        pltpu.make_async_copy(user_tt.at[:, pl.ds(0, 128)], bufU, sem_u).wait()
        pltpu.make_async_copy(item_tt.at[:, pl.ds(0, 128)], bufI, sem_u).wait()
        pltpu.make_async_copy(b1_hbm, b1_v, sem_u).wait()
        pltpu.make_async_copy(w1_hbm, w1_v, sem_w1).wait()
        ulane = jnp.full((16,), ivec[0] & 127, jnp.int32)
        ilane = jnp.full((16,), ivec2[0] & 127, jnp.int32)
        h = tuple(b1_v[pl.ds(o * 16, 16)] for o in range(4))
        for base, buf, lanev in ((0, bufU, ulane), (64, bufI, ilane)):
            def mvui(tt, hh, _buf=buf, _lanev=lanev, _base=base):
                ev = plsc.load_gather(_buf, [iota + tt * 16, _lanev])
                for j in range(16):
                    sval = ev[j]
                    kvec = jnp.full((16,), _base + tt * 16 + j, jnp.int32)
                    hh = tuple(
                        hh[o] + sval * plsc.load_gather(w1_v, [iota + (o * 16), kvec])
                        for o in range(4)
                    )
                return hh

            h = lax.fori_loop(0, 4, mvui, h)
        for o in range(4):
            parts_v[pl.ds(o * 16, 16)] = h[o]

    pltpu.sync_copy(parts_v, spart.at[s])
    plsc.subcore_barrier()

    @pl.when(shipper)
    def _core1_reduce():
        pltpu.sync_copy(spart, allp_v)
        acc = [jnp.zeros((16,), jnp.float32) for _ in range(4)]
        for j in range(16):
            for i in range(4):
                acc[i] = acc[i] + allp_v[j, pl.ds(i * 16, 16)]
        for i in range(4):
            parts_v[pl.ds(i * 16, 16)] = acc[i]
        pltpu.sync_copy(parts_v, x1_hbm)
        pltpu.semaphore_signal(sem_x, 1, core_index=0)

    @pl.when(finisher)
    def _finish_phase():
        pltpu.sync_copy(spart, allp_v)
        acc = [jnp.zeros((16,), jnp.float32) for _ in range(4)]
        for j in range(16):
            for i in range(4):
                acc[i] = acc[i] + allp_v[j, pl.ds(i * 16, 16)]
        pl.semaphore_wait(sem_x, 1)
        pltpu.sync_copy(x1_hbm, c1p_v)
        for i in range(4):
            h1_v[pl.ds(i * 16, 16)] = jnp.maximum(
                acc[i] + c1p_v[pl.ds(i * 16, 16)], 0.0
            )
        for src, dst in w2copies:
            pltpu.make_async_copy(src, dst, sem_w2).wait()
        pltpu.make_async_copy(b3_hbm, b3f_v.at[pl.ds(0, 1)], sem_w2).wait()

        def l2(tt, a2):
            hvec = h1_v[pl.ds(tt * 16, 16)]
            for j in range(16):
                sval = hvec[j]
                kvec = jnp.full((16,), tt * 16 + j, jnp.int32)
                a2 = tuple(
                    a2[i] + sval * plsc.load_gather(w2_v, [iota + (i * 16), kvec])
                    for i in range(2)
                )
            return a2

        acc2 = lax.fori_loop(
            0, 4, l2, tuple(b2_v[pl.ds(j * 16, 16)] for j in range(2))
        )
        h2a = jnp.maximum(acc2[0], 0.0)
        h2b = jnp.maximum(acc2[1], 0.0)

        p = h2a * w3_v[0, pl.ds(0, 16)] + h2b * w3_v[0, pl.ds(16, 16)]
        z = b3f_v[...][0]
        for j in range(16):
            z = z + p[j]
        zv = jnp.full((16,), z, jnp.float32)
        out_v[...] = 1.0 / (1.0 + jnp.exp(-zv))
        pltpu.sync_copy(out_v, out_hbm)


_sc_kernel = functools.partial(
    pl.kernel,
    out_type=(
        jax.ShapeDtypeStruct((16,), jnp.float32),
        jax.ShapeDtypeStruct((D,), jnp.float32),
    ),
    mesh=_mesh,
    compiler_params=pltpu.CompilerParams(
        use_tc_tiling_on_sc=True, needs_layout_passes=False
    ),
    scratch_types=[
        pltpu.VMEM((48,), jnp.int32),
        pltpu.VMEM((D, 128), jnp.float32),
        pltpu.VMEM((D, 128), jnp.float32),
        pltpu.VMEM((D, 128), jnp.float32),
        pltpu.VMEM((D, 128), jnp.float32),
        pltpu.VMEM((D, 128), jnp.float32),
        pltpu.VMEM((D, 128), jnp.float32),
        pltpu.VMEM((D, 128), jnp.float32),
        pltpu.VMEM((D, 128), jnp.float32),
        pltpu.VMEM((D, 128), jnp.float32),
        pltpu.VMEM((D, 128), jnp.float32),
        pltpu.VMEM((D,), jnp.float32),
        pltpu.VMEM((16, D), jnp.float32),
        pltpu.VMEM((D,), jnp.float32),
        pltpu.VMEM((64, 192), jnp.float32),
        pltpu.VMEM((64,), jnp.float32),
        pltpu.VMEM((32, 64), jnp.float32),
        pltpu.VMEM((32,), jnp.float32),
        pltpu.VMEM((1, 32), jnp.float32),
        pltpu.VMEM((16,), jnp.float32),
        pltpu.VMEM((64,), jnp.float32),
        pltpu.VMEM((16,), jnp.float32),
        pltpu.VMEM_SHARED((16, D), jnp.float32),
        pltpu.SemaphoreType.DMA,
        pltpu.SemaphoreType.DMA,
        pltpu.SemaphoreType.DMA,
        pltpu.SemaphoreType.DMA,
        pltpu.SemaphoreType.REGULAR,
    ],
)(_sc_body)


def kernel(user_id, item_history, item_id, user_table, item_table, W1, b1, W2, b2, W3, b3):
    out16, _ = _sc_kernel(
        user_id.astype(jnp.int32), item_id.astype(jnp.int32),
        item_history.astype(jnp.int32),
        user_table.T, item_table.T,
        W1, b1, W2, b2, W3, b3,
    )
    return out16[0].reshape(1, 1, 1)
